# baseline (speedup 1.0000x reference)
import jax
import jax.numpy as jnp
from jax import lax
from jax.experimental import pallas as pl
from jax.experimental.pallas import tpu as pltpu
from jax.experimental.pallas import tpu_sc as plsc

NU = 25000
NI = 25000
D = 64
E = 800000

HALF = 25088
DUMMY = 25080
PADN = 2 * HALF
C = 128
SLAB = 4
NBUF = 3

EROWS = 6272
WROWS = EROWS // 32
PASS = 28
RCAP = WROWS + SLAB
RTOT = 32 * RCAP
HROWS = HALF // 16


def _prep_body(dstm, srcm, wm, dl, sl_, wl, counts,
               dbuf, sbuf, wbuf, od0, os0, ow0, od1, os1, ow1, cbuf):
    c = lax.axis_index("c")
    s = lax.axis_index("s")
    w = s * 2 + c
    lanes = jnp.arange(16, dtype=jnp.int32)

    def prefill(nrows):
        def row(r, carry):
            for l in range(8):
                sl = pl.ds(l * 16, 16)
                od0[r, sl] = jnp.full((16,), DUMMY, jnp.int32)
                os0[r, sl] = jnp.zeros((16,), jnp.int32)
                ow0[r, sl] = jnp.zeros((16,), jnp.float32)
                od1[r, sl] = jnp.full((16,), DUMMY, jnp.int32)
                os1[r, sl] = jnp.zeros((16,), jnp.int32)
                ow1[r, sl] = jnp.zeros((16,), jnp.float32)
            return carry
        lax.fori_loop(0, nrows, row, 0)

    def do_pass(p, offs):
        roff0, roff1 = offs
        r0 = w * WROWS + p * PASS
        pltpu.sync_copy(dstm.at[pl.ds(r0, PASS)], dbuf)
        pltpu.sync_copy(srcm.at[pl.ds(r0, PASS)], sbuf)
        pltpu.sync_copy(wm.at[pl.ds(r0, PASS)], wbuf)
        prefill(PASS)

        def group(g, carry):
            off0, off1 = carry
            r = g // 8
            sl = pl.ds((g % 8) * 16, 16)
            dv = dbuf[r, sl]
            sv = sbuf[r, sl]
            wv = wbuf[r, sl]
            sv = jnp.where(sv >= NU, sv + (HALF - NU), sv)

            m0 = dv < NU
            mi0 = jnp.where(m0, 1, 0)
            cs0 = plsc.cumsum(mi0)
            pos0 = off0 + cs0 - mi0
            plsc.store_scatter(od0, [pos0 >> 7, pos0 & 127], dv, mask=m0)
            plsc.store_scatter(os0, [pos0 >> 7, pos0 & 127], sv, mask=m0)
            plsc.store_scatter(ow0, [pos0 >> 7, pos0 & 127], wv, mask=m0)

            m1 = dv >= NU
            mi1 = jnp.where(m1, 1, 0)
            cs1 = plsc.cumsum(mi1)
            pos1 = off1 + cs1 - mi1
            dv1 = dv - NU
            plsc.store_scatter(od1, [pos1 >> 7, pos1 & 127], dv1, mask=m1)
            plsc.store_scatter(os1, [pos1 >> 7, pos1 & 127], sv, mask=m1)
            plsc.store_scatter(ow1, [pos1 >> 7, pos1 & 127], wv, mask=m1)
            return (off0 + cs0[15], off1 + cs1[15])

        off0, off1 = lax.fori_loop(0, PASS * 8, group, (0, 0))

        base0 = w * RCAP + roff0
        pltpu.sync_copy(od0, dl.at[0, pl.ds(base0, PASS)])
        pltpu.sync_copy(os0, sl_.at[0, pl.ds(base0, PASS)])
        pltpu.sync_copy(ow0, wl.at[0, pl.ds(base0, PASS)])
        base1 = w * RCAP + roff1
        pltpu.sync_copy(od1, dl.at[1, pl.ds(base1, PASS)])
        pltpu.sync_copy(os1, sl_.at[1, pl.ds(base1, PASS)])
        pltpu.sync_copy(ow1, wl.at[1, pl.ds(base1, PASS)])
        return (roff0 + ((off0 + 127) >> 7), roff1 + ((off1 + 127) >> 7))

    roff0, roff1 = lax.fori_loop(0, WROWS // PASS, do_pass, (0, 0))

    prefill(SLAB)
    pltpu.sync_copy(od0.at[pl.ds(0, SLAB)], dl.at[0, pl.ds(w * RCAP + roff0, SLAB)])
    pltpu.sync_copy(os0.at[pl.ds(0, SLAB)], sl_.at[0, pl.ds(w * RCAP + roff0, SLAB)])
    pltpu.sync_copy(ow0.at[pl.ds(0, SLAB)], wl.at[0, pl.ds(w * RCAP + roff0, SLAB)])
    pltpu.sync_copy(od1.at[pl.ds(0, SLAB)], dl.at[1, pl.ds(w * RCAP + roff1, SLAB)])
    pltpu.sync_copy(os1.at[pl.ds(0, SLAB)], sl_.at[1, pl.ds(w * RCAP + roff1, SLAB)])
    pltpu.sync_copy(ow1.at[pl.ds(0, SLAB)], wl.at[1, pl.ds(w * RCAP + roff1, SLAB)])

    n0 = jnp.maximum(1, (roff0 + SLAB - 1) >> 2)
    n1 = jnp.maximum(1, (roff1 + SLAB - 1) >> 2)
    cbuf[...] = jnp.where(lanes == 0, n0, 0)
    pltpu.sync_copy(cbuf, counts.at[0, w])
    cbuf[...] = jnp.where(lanes == 0, n1, 0)
    pltpu.sync_copy(cbuf, counts.at[1, w])


@jax.jit
def _prep(dstm, srcm, wm):
    mesh = plsc.VectorSubcoreMesh(core_axis_name="c", subcore_axis_name="s")
    return pl.kernel(
        _prep_body,
        out_type=(
            jax.ShapeDtypeStruct((2, RTOT, C), jnp.int32),
            jax.ShapeDtypeStruct((2, RTOT, C), jnp.int32),
            jax.ShapeDtypeStruct((2, RTOT, C), jnp.float32),
            jax.ShapeDtypeStruct((2, 32, 16), jnp.int32),
        ),
        mesh=mesh,
        scratch_types=[
            pltpu.VMEM((PASS, C), jnp.int32),
            pltpu.VMEM((PASS, C), jnp.int32),
            pltpu.VMEM((PASS, C), jnp.float32),
            pltpu.VMEM((PASS, C), jnp.int32),
            pltpu.VMEM((PASS, C), jnp.int32),
            pltpu.VMEM((PASS, C), jnp.float32),
            pltpu.VMEM((PASS, C), jnp.int32),
            pltpu.VMEM((PASS, C), jnp.int32),
            pltpu.VMEM((PASS, C), jnp.float32),
            pltpu.VMEM((16,), jnp.int32),
        ],
        compiler_params=pltpu.CompilerParams(use_tc_tiling_on_sc=False,
                                             needs_layout_passes=False),
    )(dstm, srcm, wm)


def _propagate_body(emb, dl, sl_, wl, counts, zeros, out,
                    dsl, ssl, wsl, rbuf, cbuf, acc, isem, gsem, ssem):
    c = lax.axis_index("c")
    s = lax.axis_index("s")

    pltpu.sync_copy(zeros.at[pl.ds(s * HROWS, HROWS)],
                    acc.at[pl.ds(s * HROWS, HROWS)])
    plsc.subcore_barrier()

    def drain_g():
        pltpu.make_async_copy(emb.at[pl.ds(0, C)], rbuf.at[0], gsem).wait()

    def drain_s():
        pltpu.make_async_copy(emb.at[pl.ds(0, C)], rbuf.at[0], ssem).wait()

    def drain_i():
        pltpu.make_async_copy(dl.at[0, pl.ds(0, SLAB)], dsl.at[0], isem).wait()
        pltpu.make_async_copy(sl_.at[0, pl.ds(0, SLAB)], ssl.at[0], isem).wait()
        pltpu.make_async_copy(wl.at[0, pl.ds(0, SLAB)], wsl.at[0], isem).wait()

    def run_region(reg):
        base = reg * RCAP
        pltpu.sync_copy(counts.at[c, reg], cbuf)
        nslab = jnp.int32(26)

        def stage(j, buf):
            r = base + j * SLAB
            pltpu.async_copy(dl.at[c, pl.ds(r, SLAB)], dsl.at[buf], isem)
            pltpu.async_copy(sl_.at[c, pl.ds(r, SLAB)], ssl.at[buf], isem)
            pltpu.async_copy(wl.at[c, pl.ds(r, SLAB)], wsl.at[buf], isem)

        stage(0, 0)
        drain_i()
        pltpu.async_copy(emb.at[ssl.at[0, 0]], rbuf.at[0], gsem)
        pltpu.async_copy(emb.at[ssl.at[0, 1]], rbuf.at[1], gsem)

        def slab(j, carry):
            @pl.when(j < nslab)
            def _():
                _slab_body(j)
            return carry

        def _slab_body(j):
            m = lax.rem(j, 2)
            nm = 1 - m
            for k in range(SLAB):
                b = lax.rem(j + k, NBUF)
                b2 = lax.rem(j + k + 2, NBUF)
                drain_g()

                def scale(g, carry2):
                    w16 = wsl[m, k, pl.ds(g * 16, 16)]
                    for e in range(16):
                        w = w16[e]
                        idx = g * 16 + e
                        for q in range(4):
                            sl = pl.ds(q * 16, 16)
                            rbuf[b, idx, sl] = rbuf[b, idx, sl] * w
                    return carry2

                lax.fori_loop(0, C // 16, scale, 0)

                pltpu.async_copy(rbuf.at[b], acc.at[pl.ds(s * HROWS, C)],
                                 ssem)

                if k == 0:
                    @pl.when(j > 0)
                    def _():
                        drain_s()

                    @pl.when(j < nslab - 1)
                    def _():
                        stage(j + 1, nm)
                else:
                    drain_s()

                if k < 2:
                    pltpu.async_copy(emb.at[ssl.at[m, k + 2]], rbuf.at[b2],
                                     gsem)
                else:
                    if k == 2:
                        @pl.when(j < nslab - 1)
                        def _():
                            drain_i()

                    @pl.when(j < nslab - 1)
                    def _():
                        pltpu.async_copy(emb.at[ssl.at[nm, k - 2]],
                                         rbuf.at[b2], gsem)

        lax.fori_loop(0, RCAP // SLAB, slab, 0)
        drain_s()

    run_region(2 * s)
    run_region(2 * s + 1)

    plsc.subcore_barrier()
    pltpu.sync_copy(acc.at[pl.ds(s * HROWS, HROWS)],
                    out.at[pl.ds(c * HALF + s * HROWS, HROWS)])


@jax.jit
def _propagate(emb, dl, sl_, wl, counts, zeros):
    mesh = plsc.VectorSubcoreMesh(core_axis_name="c", subcore_axis_name="s")
    return pl.kernel(
        _propagate_body,
        out_type=jax.ShapeDtypeStruct((PADN, D), jnp.float32),
        mesh=mesh,
        scratch_types=[
            pltpu.VMEM((2, SLAB, C), jnp.int32),
            pltpu.VMEM((2, SLAB, C), jnp.int32),
            pltpu.VMEM((2, SLAB, C), jnp.float32),
            pltpu.VMEM((NBUF, C, D), jnp.float32),
            pltpu.VMEM((16,), jnp.int32),
            pltpu.VMEM_SHARED((HALF, D), jnp.float32),
            pltpu.SemaphoreType.DMA,
            pltpu.SemaphoreType.DMA,
            pltpu.SemaphoreType.DMA,
        ],
        compiler_params=pltpu.CompilerParams(use_tc_tiling_on_sc=False),
    )(emb, dl, sl_, wl, counts, zeros)


def _mean_body(a, b, c, d, o):
    o[...] = (a[...] + b[...] + c[...] + d[...]) * 0.25


@jax.jit
def _mean4(a, b, c, d):
    blk = 1024
    spec = pl.BlockSpec((blk, D), lambda i: (i, 0))
    return pl.pallas_call(
        _mean_body,
        grid=(PADN // blk,),
        in_specs=[spec] * 4,
        out_specs=spec,
        out_shape=jax.ShapeDtypeStruct((PADN, D), jnp.float32),
    )(a, b, c, d)


def kernel(user_emb, item_emb, edge_index, edge_weight):
    pad = jnp.zeros((HALF - NU, D), jnp.float32)
    e0 = jnp.concatenate([user_emb, pad, item_emb, pad], axis=0)

    dst = edge_index[0].astype(jnp.int32)
    src = edge_index[1].astype(jnp.int32)
    epad = EROWS * C - E
    dstm = jnp.pad(dst, (0, epad)).reshape(EROWS, C)
    srcm = jnp.pad(src, (0, epad)).reshape(EROWS, C)
    wm = jnp.pad(edge_weight, (0, epad)).reshape(EROWS, C)
    zeros = jnp.zeros((HALF, D), jnp.float32)

    dl, sl_, wl, counts = _prep(dstm, srcm, wm)
    e1 = _propagate(e0, dl, sl_, wl, counts, zeros)
    e2 = _propagate(e1, dl, sl_, wl, counts, zeros)
    e3 = _propagate(e2, dl, sl_, wl, counts, zeros)
    m = _mean4(e0, e1, e2, e3)
    return (m[:NU], m[HALF:HALF + NI])


# speedup vs baseline: 1.0645x; 1.0645x over previous
import jax
import jax.numpy as jnp
from jax import lax
from jax.experimental import pallas as pl
from jax.experimental.pallas import tpu as pltpu
from jax.experimental.pallas import tpu_sc as plsc

NU = 25000
NI = 25000
D = 64
E = 800000

HALF = 25088
DUMMY = 25080
PADN = 2 * HALF
C = 128
SLAB = 4
NSLAB = 98
EPAD = 16 * NSLAB * SLAB * C
EROWS = EPAD // C
RPT = EROWS // 16
HROWS = HALF // 16
NBUF = 3


def _prep_body(dstm, srcm, dstc, srcp, dbuf, sbuf, o0, o1, o2):
    c = lax.axis_index("c")
    s = lax.axis_index("s")
    wid = s * 2 + c
    rows = EROWS // 32
    pr = 28

    def do_pass(p, carry):
        r0 = wid * rows + p * pr
        pltpu.sync_copy(dstm.at[pl.ds(r0, pr)], dbuf)
        pltpu.sync_copy(srcm.at[pl.ds(r0, pr)], sbuf)

        def row(r, carry2):
            for l in range(8):
                sl = pl.ds(l * 16, 16)
                sv = sbuf[r, sl]
                o2[r, sl] = jnp.where(sv >= NU, sv + (HALF - NU), sv)
                dv = dbuf[r, sl]
                o0[r, sl] = jnp.where((dv >= 0) & (dv < NU), dv, DUMMY)
                dv1 = dv - NU
                o1[r, sl] = jnp.where((dv1 >= 0) & (dv1 < NU), dv1, DUMMY)
            return carry2

        lax.fori_loop(0, pr, row, 0)
        pltpu.sync_copy(o0, dstc.at[0, pl.ds(r0, pr)])
        pltpu.sync_copy(o1, dstc.at[1, pl.ds(r0, pr)])
        pltpu.sync_copy(o2, srcp.at[pl.ds(r0, pr)])
        return carry

    lax.fori_loop(0, rows // pr, do_pass, 0)


@jax.jit
def _prep(dstm, srcm):
    mesh = plsc.VectorSubcoreMesh(core_axis_name="c", subcore_axis_name="s")
    return pl.kernel(
        _prep_body,
        out_type=(
            jax.ShapeDtypeStruct((2, EROWS, C), jnp.int32),
            jax.ShapeDtypeStruct((EROWS, C), jnp.int32),
        ),
        mesh=mesh,
        scratch_types=[
            pltpu.VMEM((28, C), jnp.int32),
            pltpu.VMEM((28, C), jnp.int32),
            pltpu.VMEM((28, C), jnp.int32),
            pltpu.VMEM((28, C), jnp.int32),
            pltpu.VMEM((28, C), jnp.int32),
        ],
        compiler_params=pltpu.CompilerParams(use_tc_tiling_on_sc=False),
    )(dstm, srcm)


def _propagate_body(emb, dstc, srcp, wm, zeros, out,
                    dsl, ssl, wsl, rbuf, acc, isem, gsem, ssem):
    c = lax.axis_index("c")
    s = lax.axis_index("s")

    pltpu.sync_copy(zeros.at[pl.ds(s * HROWS, HROWS)],
                    acc.at[pl.ds(s * HROWS, HROWS)])
    plsc.subcore_barrier()

    row0 = s * RPT

    def drain_g():
        pltpu.make_async_copy(emb.at[pl.ds(0, C)], rbuf.at[0], gsem).wait()

    def drain_s():
        pltpu.make_async_copy(emb.at[pl.ds(0, C)], rbuf.at[0], ssem).wait()

    def drain_i():
        pltpu.make_async_copy(dstc.at[0, pl.ds(0, SLAB)], dsl.at[0], isem).wait()
        pltpu.make_async_copy(srcp.at[pl.ds(0, SLAB)], ssl.at[0], isem).wait()
        pltpu.make_async_copy(wm.at[pl.ds(0, SLAB)], wsl.at[0], isem).wait()

    def stage(j, buf):
        r = row0 + j * SLAB
        pltpu.async_copy(dstc.at[c, pl.ds(r, SLAB)], dsl.at[buf], isem)
        pltpu.async_copy(srcp.at[pl.ds(r, SLAB)], ssl.at[buf], isem)
        pltpu.async_copy(wm.at[pl.ds(r, SLAB)], wsl.at[buf], isem)

    stage(0, 0)
    drain_i()
    pltpu.async_copy(emb.at[ssl.at[0, 0]], rbuf.at[0], gsem)
    pltpu.async_copy(emb.at[ssl.at[0, 1]], rbuf.at[1], gsem)

    def slab(j, carry):
        m = lax.rem(j, 2)
        nm = 1 - m
        for k in range(SLAB):
            b = lax.rem(j + k, NBUF)
            b2 = lax.rem(j + k + 2, NBUF)
            drain_g()

            def scale(g, carry2):
                w16 = wsl[m, k, pl.ds(g * 16, 16)]
                for e in range(16):
                    w = w16[e]
                    idx = g * 16 + e
                    for q in range(4):
                        sl = pl.ds(q * 16, 16)
                        rbuf[b, idx, sl] = rbuf[b, idx, sl] * w
                return carry2

            lax.fori_loop(0, C // 16, scale, 0, unroll=4)

            pltpu.async_copy(rbuf.at[b], acc.at[dsl.at[m, k]], ssem, add=True)

            if k == 0:
                @pl.when(j > 0)
                def _():
                    drain_s()

                @pl.when(j < NSLAB - 1)
                def _():
                    stage(j + 1, nm)
            else:
                drain_s()

            if k < 2:
                pltpu.async_copy(emb.at[ssl.at[m, k + 2]], rbuf.at[b2], gsem)
            else:
                if k == 2:
                    @pl.when(j < NSLAB - 1)
                    def _():
                        drain_i()

                @pl.when(j < NSLAB - 1)
                def _():
                    pltpu.async_copy(emb.at[ssl.at[nm, k - 2]],
                                     rbuf.at[b2], gsem)
        return carry

    lax.fori_loop(0, NSLAB, slab, 0)
    drain_s()

    plsc.subcore_barrier()
    pltpu.sync_copy(acc.at[pl.ds(s * HROWS, HROWS)],
                    out.at[pl.ds(c * HALF + s * HROWS, HROWS)])


@jax.jit
def _propagate(emb, dstc, srcp, wm, zeros):
    mesh = plsc.VectorSubcoreMesh(core_axis_name="c", subcore_axis_name="s")
    return pl.kernel(
        _propagate_body,
        out_type=jax.ShapeDtypeStruct((PADN, D), jnp.float32),
        mesh=mesh,
        scratch_types=[
            pltpu.VMEM((2, SLAB, C), jnp.int32),
            pltpu.VMEM((2, SLAB, C), jnp.int32),
            pltpu.VMEM((2, SLAB, C), jnp.float32),
            pltpu.VMEM((NBUF, C, D), jnp.float32),
            pltpu.VMEM_SHARED((HALF, D), jnp.float32),
            pltpu.SemaphoreType.DMA,
            pltpu.SemaphoreType.DMA,
            pltpu.SemaphoreType.DMA,
        ],
        compiler_params=pltpu.CompilerParams(use_tc_tiling_on_sc=False),
    )(emb, dstc, srcp, wm, zeros)


def _mean_body(a, b, c, d, o):
    o[...] = (a[...] + b[...] + c[...] + d[...]) * 0.25


@jax.jit
def _mean4(a, b, c, d):
    blk = 1024
    spec = pl.BlockSpec((blk, D), lambda i: (i, 0))
    return pl.pallas_call(
        _mean_body,
        grid=(PADN // blk,),
        in_specs=[spec] * 4,
        out_specs=spec,
        out_shape=jax.ShapeDtypeStruct((PADN, D), jnp.float32),
    )(a, b, c, d)


def kernel(user_emb, item_emb, edge_index, edge_weight):
    pad = jnp.zeros((HALF - NU, D), jnp.float32)
    e0 = jnp.concatenate([user_emb, pad, item_emb, pad], axis=0)

    dst = edge_index[0].astype(jnp.int32)
    src = edge_index[1].astype(jnp.int32)
    epad = EPAD - E
    dstm = jnp.pad(dst, (0, epad)).reshape(EROWS, C)
    srcm = jnp.pad(src, (0, epad)).reshape(EROWS, C)
    wm = jnp.pad(edge_weight, (0, epad)).reshape(EROWS, C)
    zeros = jnp.zeros((HALF, D), jnp.float32)

    dstc, srcp = _prep(dstm, srcm)
    e1 = _propagate(e0, dstc, srcp, wm, zeros)
    e2 = _propagate(e1, dstc, srcp, wm, zeros)
    e3 = _propagate(e2, dstc, srcp, wm, zeros)
    m = _mean4(e0, e1, e2, e3)
    return (m[:NU], m[HALF:HALF + NI])


# speedup vs baseline: 1.1126x; 1.0451x over previous
import jax
import jax.numpy as jnp
from jax import lax
from jax.experimental import pallas as pl
from jax.experimental.pallas import tpu as pltpu
from jax.experimental.pallas import tpu_sc as plsc

NU = 25000
NI = 25000
D = 64
E = 800000

HALF = 25088
DUMMY = 25080
PADN = 2 * HALF
C = 128
SLAB = 4
NSLAB = 98
EPAD = 16 * NSLAB * SLAB * C
EROWS = EPAD // C
RPT = EROWS // 16
HROWS = HALF // 16
NBUF = 3


def _prep_body(dstm, srcm, dstc, srcp, dbuf, sbuf, o0, o1, o2):
    c = lax.axis_index("c")
    s = lax.axis_index("s")
    wid = s * 2 + c
    rows = EROWS // 32
    pr = 28

    def do_pass(p, carry):
        r0 = wid * rows + p * pr
        pltpu.sync_copy(dstm.at[pl.ds(r0, pr)], dbuf)
        pltpu.sync_copy(srcm.at[pl.ds(r0, pr)], sbuf)

        def row(r, carry2):
            for l in range(8):
                sl = pl.ds(l * 16, 16)
                sv = sbuf[r, sl]
                o2[r, sl] = jnp.where(sv >= NU, sv + (HALF - NU), sv)
                dv = dbuf[r, sl]
                o0[r, sl] = jnp.where((dv >= 0) & (dv < NU), dv, DUMMY)
                dv1 = dv - NU
                o1[r, sl] = jnp.where((dv1 >= 0) & (dv1 < NU), dv1, DUMMY)
            return carry2

        lax.fori_loop(0, pr, row, 0)
        pltpu.sync_copy(o0, dstc.at[0, pl.ds(r0, pr)])
        pltpu.sync_copy(o1, dstc.at[1, pl.ds(r0, pr)])
        pltpu.sync_copy(o2, srcp.at[pl.ds(r0, pr)])
        return carry

    lax.fori_loop(0, rows // pr, do_pass, 0)


@jax.jit
def _prep(dstm, srcm):
    mesh = plsc.VectorSubcoreMesh(core_axis_name="c", subcore_axis_name="s")
    return pl.kernel(
        _prep_body,
        out_type=(
            jax.ShapeDtypeStruct((2, EROWS, C), jnp.int32),
            jax.ShapeDtypeStruct((EROWS, C), jnp.int32),
        ),
        mesh=mesh,
        scratch_types=[
            pltpu.VMEM((28, C), jnp.int32),
            pltpu.VMEM((28, C), jnp.int32),
            pltpu.VMEM((28, C), jnp.int32),
            pltpu.VMEM((28, C), jnp.int32),
            pltpu.VMEM((28, C), jnp.int32),
        ],
        compiler_params=pltpu.CompilerParams(use_tc_tiling_on_sc=False),
    )(dstm, srcm)


def _propagate_body(emb, dstc, srcp, wm, zeros, out,
                    dsl, ssl, wsl, rbuf, acc, isem, gsem, ssem):
    c = lax.axis_index("c")
    s = lax.axis_index("s")

    pltpu.sync_copy(zeros.at[pl.ds(s * HROWS, HROWS)],
                    acc.at[pl.ds(s * HROWS, HROWS)])
    plsc.subcore_barrier()

    row0 = s * RPT

    def drain_g():
        pltpu.make_async_copy(emb.at[pl.ds(0, C)], rbuf.at[0], gsem).wait()

    def drain_s():
        pltpu.make_async_copy(emb.at[pl.ds(0, C)], rbuf.at[0], ssem).wait()

    def drain_i():
        pltpu.make_async_copy(dstc.at[0, pl.ds(0, SLAB)], dsl.at[0], isem).wait()
        pltpu.make_async_copy(srcp.at[pl.ds(0, SLAB)], ssl.at[0], isem).wait()
        pltpu.make_async_copy(wm.at[pl.ds(0, SLAB)], wsl.at[0], isem).wait()

    def stage(j, buf):
        r = row0 + j * SLAB
        pltpu.async_copy(dstc.at[c, pl.ds(r, SLAB)], dsl.at[buf], isem)
        pltpu.async_copy(srcp.at[pl.ds(r, SLAB)], ssl.at[buf], isem)
        pltpu.async_copy(wm.at[pl.ds(r, SLAB)], wsl.at[buf], isem)

    stage(0, 0)
    drain_i()
    pltpu.async_copy(emb.at[ssl.at[0, 0]], rbuf.at[0], gsem)
    pltpu.async_copy(emb.at[ssl.at[0, 1]], rbuf.at[1], gsem)

    def slab(j, carry):
        m = lax.rem(j, 2)
        nm = 1 - m
        for k in range(SLAB):
            b = lax.rem(j + k, NBUF)
            b2 = lax.rem(j + k + 2, NBUF)
            drain_g()

            def scale(g, carry2):
                w16 = wsl[m, k, pl.ds(g * 16, 16)]
                for e in range(16):
                    w = w16[e]
                    idx = g * 16 + e
                    for q in range(4):
                        sl = pl.ds(q * 16, 16)
                        rbuf[b, idx, sl] = rbuf[b, idx, sl] * w
                return carry2

            lax.fori_loop(0, C // 16, scale, 0, unroll=2)

            pltpu.async_copy(rbuf.at[b], acc.at[dsl.at[m, k]], ssem, add=True)

            if k == 0:
                @pl.when(j > 0)
                def _():
                    drain_s()

                @pl.when(j < NSLAB - 1)
                def _():
                    stage(j + 1, nm)
            else:
                drain_s()

            if k < 2:
                pltpu.async_copy(emb.at[ssl.at[m, k + 2]], rbuf.at[b2], gsem)
            else:
                if k == 2:
                    @pl.when(j < NSLAB - 1)
                    def _():
                        drain_i()

                @pl.when(j < NSLAB - 1)
                def _():
                    pltpu.async_copy(emb.at[ssl.at[nm, k - 2]],
                                     rbuf.at[b2], gsem)
        return carry

    lax.fori_loop(0, NSLAB, slab, 0)
    drain_s()

    plsc.subcore_barrier()
    pltpu.sync_copy(acc.at[pl.ds(s * HROWS, HROWS)],
                    out.at[pl.ds(c * HALF + s * HROWS, HROWS)])


@jax.jit
def _propagate(emb, dstc, srcp, wm, zeros):
    mesh = plsc.VectorSubcoreMesh(core_axis_name="c", subcore_axis_name="s")
    return pl.kernel(
        _propagate_body,
        out_type=jax.ShapeDtypeStruct((PADN, D), jnp.float32),
        mesh=mesh,
        scratch_types=[
            pltpu.VMEM((2, SLAB, C), jnp.int32),
            pltpu.VMEM((2, SLAB, C), jnp.int32),
            pltpu.VMEM((2, SLAB, C), jnp.float32),
            pltpu.VMEM((NBUF, C, D), jnp.float32),
            pltpu.VMEM_SHARED((HALF, D), jnp.float32),
            pltpu.SemaphoreType.DMA,
            pltpu.SemaphoreType.DMA,
            pltpu.SemaphoreType.DMA,
        ],
        compiler_params=pltpu.CompilerParams(use_tc_tiling_on_sc=False),
    )(emb, dstc, srcp, wm, zeros)


def _mean_body(a, b, c, d, o):
    o[...] = (a[...] + b[...] + c[...] + d[...]) * 0.25


@jax.jit
def _mean4(a, b, c, d):
    blk = 1024
    spec = pl.BlockSpec((blk, D), lambda i: (i, 0))
    return pl.pallas_call(
        _mean_body,
        grid=(PADN // blk,),
        in_specs=[spec] * 4,
        out_specs=spec,
        out_shape=jax.ShapeDtypeStruct((PADN, D), jnp.float32),
    )(a, b, c, d)


def kernel(user_emb, item_emb, edge_index, edge_weight):
    pad = jnp.zeros((HALF - NU, D), jnp.float32)
    e0 = jnp.concatenate([user_emb, pad, item_emb, pad], axis=0)

    dst = edge_index[0].astype(jnp.int32)
    src = edge_index[1].astype(jnp.int32)
    epad = EPAD - E
    dstm = jnp.pad(dst, (0, epad)).reshape(EROWS, C)
    srcm = jnp.pad(src, (0, epad)).reshape(EROWS, C)
    wm = jnp.pad(edge_weight, (0, epad)).reshape(EROWS, C)
    zeros = jnp.zeros((HALF, D), jnp.float32)

    dstc, srcp = _prep(dstm, srcm)
    e1 = _propagate(e0, dstc, srcp, wm, zeros)
    e2 = _propagate(e1, dstc, srcp, wm, zeros)
    e3 = _propagate(e2, dstc, srcp, wm, zeros)
    m = _mean4(e0, e1, e2, e3)
    return (m[:NU], m[HALF:HALF + NI])


# speedup vs baseline: 1.1427x; 1.0271x over previous
import jax
import jax.numpy as jnp
from jax import lax
from jax.experimental import pallas as pl
from jax.experimental.pallas import tpu as pltpu
from jax.experimental.pallas import tpu_sc as plsc

NU = 25000
NI = 25000
D = 64
E = 800000

HALF = 25088
DUMMY = 25080
PADN = 2 * HALF
C = 128
SLAB = 6
NSLAB = 66
EPAD = 16 * NSLAB * SLAB * C
EROWS = EPAD // C
RPT = EROWS // 16
HROWS = HALF // 16
NBUF = 3


def _prep_body(dstm, srcm, dstc, srcp, dbuf, sbuf, o0, o1, o2):
    c = lax.axis_index("c")
    s = lax.axis_index("s")
    wid = s * 2 + c
    rows = EROWS // 32
    pr = 18

    def do_pass(p, carry):
        r0 = wid * rows + p * pr
        pltpu.sync_copy(dstm.at[pl.ds(r0, pr)], dbuf)
        pltpu.sync_copy(srcm.at[pl.ds(r0, pr)], sbuf)

        def row(r, carry2):
            for l in range(8):
                sl = pl.ds(l * 16, 16)
                sv = sbuf[r, sl]
                o2[r, sl] = jnp.where(sv >= NU, sv + (HALF - NU), sv)
                dv = dbuf[r, sl]
                o0[r, sl] = jnp.where((dv >= 0) & (dv < NU), dv, DUMMY)
                dv1 = dv - NU
                o1[r, sl] = jnp.where((dv1 >= 0) & (dv1 < NU), dv1, DUMMY)
            return carry2

        lax.fori_loop(0, pr, row, 0)
        pltpu.sync_copy(o0, dstc.at[0, pl.ds(r0, pr)])
        pltpu.sync_copy(o1, dstc.at[1, pl.ds(r0, pr)])
        pltpu.sync_copy(o2, srcp.at[pl.ds(r0, pr)])
        return carry

    lax.fori_loop(0, rows // pr, do_pass, 0)


@jax.jit
def _prep(dstm, srcm):
    mesh = plsc.VectorSubcoreMesh(core_axis_name="c", subcore_axis_name="s")
    return pl.kernel(
        _prep_body,
        out_type=(
            jax.ShapeDtypeStruct((2, EROWS, C), jnp.int32),
            jax.ShapeDtypeStruct((EROWS, C), jnp.int32),
        ),
        mesh=mesh,
        scratch_types=[
            pltpu.VMEM((18, C), jnp.int32),
            pltpu.VMEM((18, C), jnp.int32),
            pltpu.VMEM((18, C), jnp.int32),
            pltpu.VMEM((18, C), jnp.int32),
            pltpu.VMEM((18, C), jnp.int32),
        ],
        compiler_params=pltpu.CompilerParams(use_tc_tiling_on_sc=False),
    )(dstm, srcm)


def _propagate_body(emb, dstc, srcp, wm, zeros, out,
                    dsl, ssl, wsl, rbuf, acc, isem, gsem, ssem):
    c = lax.axis_index("c")
    s = lax.axis_index("s")

    pltpu.sync_copy(zeros.at[pl.ds(s * HROWS, HROWS)],
                    acc.at[pl.ds(s * HROWS, HROWS)])
    plsc.subcore_barrier()

    row0 = s * RPT

    def drain_g():
        pltpu.make_async_copy(emb.at[pl.ds(0, C)], rbuf.at[0], gsem).wait()

    def drain_s():
        pltpu.make_async_copy(emb.at[pl.ds(0, C)], rbuf.at[0], ssem).wait()

    def drain_i():
        pltpu.make_async_copy(dstc.at[0, pl.ds(0, SLAB)], dsl.at[0], isem).wait()
        pltpu.make_async_copy(srcp.at[pl.ds(0, SLAB)], ssl.at[0], isem).wait()
        pltpu.make_async_copy(wm.at[pl.ds(0, SLAB)], wsl.at[0], isem).wait()

    def stage(j, buf):
        r = row0 + j * SLAB
        pltpu.async_copy(dstc.at[c, pl.ds(r, SLAB)], dsl.at[buf], isem)
        pltpu.async_copy(srcp.at[pl.ds(r, SLAB)], ssl.at[buf], isem)
        pltpu.async_copy(wm.at[pl.ds(r, SLAB)], wsl.at[buf], isem)

    stage(0, 0)
    drain_i()
    pltpu.async_copy(emb.at[ssl.at[0, 0]], rbuf.at[0], gsem)
    pltpu.async_copy(emb.at[ssl.at[0, 1]], rbuf.at[1], gsem)

    def slab(j, carry):
        m = lax.rem(j, 2)
        nm = 1 - m
        for k in range(SLAB):
            b = k % NBUF
            b2 = (k + 2) % NBUF
            drain_g()

            def scale(g, carry2):
                w16 = wsl[m, k, pl.ds(g * 16, 16)]
                for e in range(16):
                    w = w16[e]
                    idx = g * 16 + e
                    for q in range(4):
                        sl = pl.ds(q * 16, 16)
                        rbuf[b, idx, sl] = rbuf[b, idx, sl] * w
                return carry2

            lax.fori_loop(0, C // 16, scale, 0)

            pltpu.async_copy(rbuf.at[b], acc.at[dsl.at[m, k]], ssem, add=True)

            if k == 0:
                @pl.when(j > 0)
                def _():
                    drain_s()

                @pl.when(j < NSLAB - 1)
                def _():
                    stage(j + 1, nm)
            else:
                drain_s()

            if k < SLAB - 2:
                pltpu.async_copy(emb.at[ssl.at[m, k + 2]], rbuf.at[b2], gsem)
            else:
                if k == SLAB - 2:
                    @pl.when(j < NSLAB - 1)
                    def _():
                        drain_i()

                @pl.when(j < NSLAB - 1)
                def _():
                    pltpu.async_copy(emb.at[ssl.at[nm, k - (SLAB - 2)]],
                                     rbuf.at[b2], gsem)
        return carry

    lax.fori_loop(0, NSLAB, slab, 0)
    drain_s()

    plsc.subcore_barrier()
    pltpu.sync_copy(acc.at[pl.ds(s * HROWS, HROWS)],
                    out.at[pl.ds(c * HALF + s * HROWS, HROWS)])


@jax.jit
def _propagate(emb, dstc, srcp, wm, zeros):
    mesh = plsc.VectorSubcoreMesh(core_axis_name="c", subcore_axis_name="s")
    return pl.kernel(
        _propagate_body,
        out_type=jax.ShapeDtypeStruct((PADN, D), jnp.float32),
        mesh=mesh,
        scratch_types=[
            pltpu.VMEM((2, SLAB, C), jnp.int32),
            pltpu.VMEM((2, SLAB, C), jnp.int32),
            pltpu.VMEM((2, SLAB, C), jnp.float32),
            pltpu.VMEM((NBUF, C, D), jnp.float32),
            pltpu.VMEM_SHARED((HALF, D), jnp.float32),
            pltpu.SemaphoreType.DMA,
            pltpu.SemaphoreType.DMA,
            pltpu.SemaphoreType.DMA,
        ],
        compiler_params=pltpu.CompilerParams(use_tc_tiling_on_sc=False),
    )(emb, dstc, srcp, wm, zeros)


def _mean_body(a, b, c, d, o):
    o[...] = (a[...] + b[...] + c[...] + d[...]) * 0.25


@jax.jit
def _mean4(a, b, c, d):
    blk = 1024
    spec = pl.BlockSpec((blk, D), lambda i: (i, 0))
    return pl.pallas_call(
        _mean_body,
        grid=(PADN // blk,),
        in_specs=[spec] * 4,
        out_specs=spec,
        out_shape=jax.ShapeDtypeStruct((PADN, D), jnp.float32),
    )(a, b, c, d)


def kernel(user_emb, item_emb, edge_index, edge_weight):
    pad = jnp.zeros((HALF - NU, D), jnp.float32)
    e0 = jnp.concatenate([user_emb, pad, item_emb, pad], axis=0)

    dst = edge_index[0].astype(jnp.int32)
    src = edge_index[1].astype(jnp.int32)
    epad = EPAD - E
    dstm = jnp.pad(dst, (0, epad)).reshape(EROWS, C)
    srcm = jnp.pad(src, (0, epad)).reshape(EROWS, C)
    wm = jnp.pad(edge_weight, (0, epad)).reshape(EROWS, C)
    zeros = jnp.zeros((HALF, D), jnp.float32)

    dstc, srcp = _prep(dstm, srcm)
    e1 = _propagate(e0, dstc, srcp, wm, zeros)
    e2 = _propagate(e1, dstc, srcp, wm, zeros)
    e3 = _propagate(e2, dstc, srcp, wm, zeros)
    m = _mean4(e0, e1, e2, e3)
    return (m[:NU], m[HALF:HALF + NI])


# speedup vs baseline: 1.1620x; 1.0168x over previous
import jax
import jax.numpy as jnp
from jax import lax
from jax.experimental import pallas as pl
from jax.experimental.pallas import tpu as pltpu
from jax.experimental.pallas import tpu_sc as plsc

NU = 25000
NI = 25000
D = 64
E = 800000

HALF = 25088
DUMMY = 25080
PADN = 2 * HALF
C = 128
SLAB = 4
NSLAB = 98
EPAD = 16 * NSLAB * SLAB * C
EROWS = EPAD // C
RPT = EROWS // 16
HROWS = HALF // 16
NBUF = 3


def _prep_body(dstm, srcm, dstc, srcp, dbuf, sbuf, o0, o1, o2):
    c = lax.axis_index("c")
    s = lax.axis_index("s")
    wid = s * 2 + c
    rows = EROWS // 32
    pr = 28

    def do_pass(p, carry):
        r0 = wid * rows + p * pr
        pltpu.sync_copy(dstm.at[pl.ds(r0, pr)], dbuf)
        pltpu.sync_copy(srcm.at[pl.ds(r0, pr)], sbuf)

        def row(r, carry2):
            for l in range(8):
                sl = pl.ds(l * 16, 16)
                sv = sbuf[r, sl]
                o2[r, sl] = jnp.where(sv >= NU, sv + (HALF - NU), sv)
                dv = dbuf[r, sl]
                o0[r, sl] = jnp.where((dv >= 0) & (dv < NU), dv, DUMMY)
                dv1 = dv - NU
                o1[r, sl] = jnp.where((dv1 >= 0) & (dv1 < NU), dv1, DUMMY)
            return carry2

        lax.fori_loop(0, pr, row, 0)
        pltpu.sync_copy(o0, dstc.at[0, pl.ds(r0, pr)])
        pltpu.sync_copy(o1, dstc.at[1, pl.ds(r0, pr)])
        pltpu.sync_copy(o2, srcp.at[pl.ds(r0, pr)])
        return carry

    lax.fori_loop(0, rows // pr, do_pass, 0)


@jax.jit
def _prep(dstm, srcm):
    mesh = plsc.VectorSubcoreMesh(core_axis_name="c", subcore_axis_name="s")
    return pl.kernel(
        _prep_body,
        out_type=(
            jax.ShapeDtypeStruct((2, EROWS, C), jnp.int32),
            jax.ShapeDtypeStruct((EROWS, C), jnp.int32),
        ),
        mesh=mesh,
        scratch_types=[
            pltpu.VMEM((28, C), jnp.int32),
            pltpu.VMEM((28, C), jnp.int32),
            pltpu.VMEM((28, C), jnp.int32),
            pltpu.VMEM((28, C), jnp.int32),
            pltpu.VMEM((28, C), jnp.int32),
        ],
        compiler_params=pltpu.CompilerParams(use_tc_tiling_on_sc=False),
    )(dstm, srcm)


def _propagate_body(emb, dstc, srcp, wm, zeros, out,
                    dsl, ssl, wsl, rbuf, acc, isem, gsem, ssem):
    c = lax.axis_index("c")
    s = lax.axis_index("s")

    pltpu.sync_copy(zeros.at[pl.ds(s * HROWS, HROWS)],
                    acc.at[pl.ds(s * HROWS, HROWS)])
    plsc.subcore_barrier()

    row0 = s * RPT

    def drain_g():
        pltpu.make_async_copy(emb.at[pl.ds(0, C)], rbuf.at[0], gsem).wait()

    def drain_s():
        pltpu.make_async_copy(emb.at[pl.ds(0, C)], rbuf.at[0], ssem).wait()

    def drain_i():
        pltpu.make_async_copy(dstc.at[0, pl.ds(0, SLAB)], dsl.at[0], isem).wait()
        pltpu.make_async_copy(srcp.at[pl.ds(0, SLAB)], ssl.at[0], isem).wait()
        pltpu.make_async_copy(wm.at[pl.ds(0, SLAB)], wsl.at[0], isem).wait()

    def stage(j, buf):
        r = row0 + j * SLAB
        pltpu.async_copy(dstc.at[c, pl.ds(r, SLAB)], dsl.at[buf], isem)
        pltpu.async_copy(srcp.at[pl.ds(r, SLAB)], ssl.at[buf], isem)
        pltpu.async_copy(wm.at[pl.ds(r, SLAB)], wsl.at[buf], isem)

    stage(0, 0)
    drain_i()
    pltpu.async_copy(emb.at[ssl.at[0, 0]], rbuf.at[0], gsem)
    pltpu.async_copy(emb.at[ssl.at[0, 1]], rbuf.at[1], gsem)

    def slab(j, carry):
        m = lax.rem(j, 2)
        nm = 1 - m
        for k in range(SLAB):
            b = lax.rem(j + k, NBUF)
            b2 = lax.rem(j + k + 2, NBUF)
            drain_g()

            def scale(g, carry2):
                w16 = wsl[m, k, pl.ds(g * 16, 16)]
                for e in range(16):
                    w = w16[e]
                    idx = g * 16 + e
                    for q in range(4):
                        sl = pl.ds(q * 16, 16)
                        rbuf[b, idx, sl] = rbuf[b, idx, sl] * w
                return carry2

            lax.fori_loop(0, C // 16, scale, 0)

            pltpu.async_copy(rbuf.at[b], acc.at[dsl.at[m, k]], ssem, add=True)

            if k == 0:
                @pl.when(j > 0)
                def _():
                    drain_s()

                @pl.when(j < NSLAB - 1)
                def _():
                    stage(j + 1, nm)
            else:
                drain_s()

            if k < 2:
                pltpu.async_copy(emb.at[ssl.at[m, k + 2]], rbuf.at[b2], gsem)
            else:
                if k == 2:
                    @pl.when(j < NSLAB - 1)
                    def _():
                        drain_i()

                @pl.when(j < NSLAB - 1)
                def _():
                    pltpu.async_copy(emb.at[ssl.at[nm, k - 2]],
                                     rbuf.at[b2], gsem)
        return carry

    lax.fori_loop(0, NSLAB, slab, 0)
    drain_s()

    plsc.subcore_barrier()
    pltpu.sync_copy(acc.at[pl.ds(s * HROWS, HROWS)],
                    out.at[pl.ds(c * HALF + s * HROWS, HROWS)])


@jax.jit
def _propagate(emb, dstc, srcp, wm, zeros):
    mesh = plsc.VectorSubcoreMesh(core_axis_name="c", subcore_axis_name="s")
    return pl.kernel(
        _propagate_body,
        out_type=jax.ShapeDtypeStruct((PADN, D), jnp.float32),
        mesh=mesh,
        scratch_types=[
            pltpu.VMEM((2, SLAB, C), jnp.int32),
            pltpu.VMEM((2, SLAB, C), jnp.int32),
            pltpu.VMEM((2, SLAB, C), jnp.float32),
            pltpu.VMEM((NBUF, C, D), jnp.float32),
            pltpu.VMEM_SHARED((HALF, D), jnp.float32),
            pltpu.SemaphoreType.DMA,
            pltpu.SemaphoreType.DMA,
            pltpu.SemaphoreType.DMA,
        ],
        compiler_params=pltpu.CompilerParams(use_tc_tiling_on_sc=False),
    )(emb, dstc, srcp, wm, zeros)


def _mean_body(a, b, c, d, o):
    o[...] = (a[...] + b[...] + c[...] + d[...]) * 0.25


@jax.jit
def _mean4(a, b, c, d):
    blk = 1024
    spec = pl.BlockSpec((blk, D), lambda i: (i, 0))
    return pl.pallas_call(
        _mean_body,
        grid=(PADN // blk,),
        in_specs=[spec] * 4,
        out_specs=spec,
        out_shape=jax.ShapeDtypeStruct((PADN, D), jnp.float32),
    )(a, b, c, d)


def kernel(user_emb, item_emb, edge_index, edge_weight):
    pad = jnp.zeros((HALF - NU, D), jnp.float32)
    e0 = jnp.concatenate([user_emb, pad, item_emb, pad], axis=0)

    dst = edge_index[0].astype(jnp.int32)
    src = edge_index[1].astype(jnp.int32)
    epad = EPAD - E
    dstm = jnp.pad(dst, (0, epad)).reshape(EROWS, C)
    srcm = jnp.pad(src, (0, epad)).reshape(EROWS, C)
    wm = jnp.pad(edge_weight, (0, epad)).reshape(EROWS, C)
    zeros = jnp.zeros((HALF, D), jnp.float32)

    dstc, srcp = _prep(dstm, srcm)
    e1 = _propagate(e0, dstc, srcp, wm, zeros)
    e2 = _propagate(e1, dstc, srcp, wm, zeros)
    e3 = _propagate(e2, dstc, srcp, wm, zeros)
    m = _mean4(e0, e1, e2, e3)
    return (m[:NU], m[HALF:HALF + NI])
